# baseline (speedup 1.0000x reference)
import functools

import jax
import jax.numpy as jnp
from jax import lax
from jax.experimental import pallas as pl
from jax.experimental.pallas import tpu as pltpu
from jax.experimental.pallas import tpu_sc as plsc

NUM = 16
H = 1024
W = 1024
NPIX = H * W

_info = plsc.get_sparse_core_info()
NC, NS, L = _info.num_cores, _info.num_subcores, _info.num_lanes
NW = NC * NS
PPW = NPIX // NW
NBINS = NUM * NUM


def _sc_hist_kernel(pred_hbm, true_hbm, hist_out, mse_out,
                    pbuf, tbuf, hacc, rowbuf, msebuf):
    wid = lax.axis_index("s") * NC + lax.axis_index("c")
    base = wid * PPW

    pltpu.sync_copy(pred_hbm.at[pl.ds(base, PPW)], pbuf)
    pltpu.sync_copy(true_hbm.at[pl.ds(base, PPW)], tbuf)

    zero16 = jnp.zeros((L,), jnp.float32)

    def zbody(i, _):
        hacc[pl.ds(pl.multiple_of(i * L, L), L)] = zero16
        return 0
    lax.fori_loop(0, (L * NBINS) // L, zbody, 0)

    lane_offs = lax.iota(jnp.int32, L) * NBINS
    ones = jnp.ones((L,), jnp.float32)

    def body(i, mseacc):
        off = pl.multiple_of(i * L, L)
        p = pbuf[pl.ds(off, L)]
        t = tbuf[pl.ds(off, L)]
        d = p - t
        comb = (p * 16.0 + t).astype(jnp.int32) + lane_offs
        plsc.addupdate_scatter(hacc, [comb], ones)
        return mseacc + d * d

    msevec = lax.fori_loop(0, PPW // L, body, zero16)

    for c in range(NBINS // L):
        def mbody(l, acc, c=c):
            return acc + hacc[pl.ds(pl.multiple_of(l * NBINS + c * L, L), L)]
        rowbuf[pl.ds(c * L, L)] = lax.fori_loop(0, L, mbody, zero16)

    msebuf[...] = msevec
    pltpu.sync_copy(rowbuf, hist_out.at[wid])
    pltpu.sync_copy(msebuf, mse_out.at[wid])


_sc_hist = functools.partial(
    pl.kernel,
    mesh=plsc.VectorSubcoreMesh(core_axis_name="c", subcore_axis_name="s"),
    out_type=[jax.ShapeDtypeStruct((NW, NBINS), jnp.float32),
              jax.ShapeDtypeStruct((NW, L), jnp.float32)],
    scratch_types=[pltpu.VMEM((PPW,), jnp.float32),
                   pltpu.VMEM((PPW,), jnp.float32),
                   pltpu.VMEM((L * NBINS,), jnp.float32),
                   pltpu.VMEM((NBINS,), jnp.float32),
                   pltpu.VMEM((L,), jnp.float32)],
    compiler_params=pltpu.CompilerParams(needs_layout_passes=False),
)(_sc_hist_kernel)


def _tc_epilogue_kernel(hist_ref, mse_ref, out_ref):
    h = hist_ref[...]
    flat = jnp.sum(h, axis=0, keepdims=True)

    bi = jax.lax.broadcasted_iota(jnp.int32, (NUM, NBINS), 0)
    bm = jax.lax.broadcasted_iota(jnp.int32, (NUM, NBINS), 1)
    d = jnp.where((bm >> 4) == bi, flat, 0.0)
    cm = jax.lax.broadcasted_iota(jnp.int32, (NBINS, NUM), 0)
    cj = jax.lax.broadcasted_iota(jnp.int32, (NBINS, NUM), 1)
    c = ((cm & (NUM - 1)) == cj).astype(jnp.float32)
    j = jax.lax.dot_general(d, c, (((1,), (0,)), ((), ())),
                            preferred_element_type=jnp.float32)

    ri = jax.lax.broadcasted_iota(jnp.int32, (NUM, NUM), 0)
    ci = jax.lax.broadcasted_iota(jnp.int32, (NUM, NUM), 1)
    valid = (ri >= 1) & (ci >= 1)
    inter = jnp.where(valid, j, 0.0)
    pc = jnp.sum(j, axis=1, keepdims=True)
    tc = jnp.sum(j, axis=0, keepdims=True)
    union = pc + tc - inter
    iou = jnp.where(valid & (union != 0.0),
                    inter / jnp.maximum(union, 1e-12), 0.0)
    max_p = jnp.max(iou, axis=1, keepdims=True)
    max_t = jnp.max(iou, axis=0, keepdims=True)
    rv = (jax.lax.broadcasted_iota(jnp.int32, (NUM, 1), 0) >= 1) & (pc > 0)
    cv = (jax.lax.broadcasted_iota(jnp.int32, (1, NUM), 1) >= 1) & (tc > 0)
    loss_p = jnp.sum(jnp.where(rv, 1.0 - max_p, 0.0))
    loss_t = jnp.sum(jnp.where(cv, 1.0 - max_t, 0.0))
    ninst = (jnp.sum(rv.astype(jnp.float32))
             + jnp.sum(cv.astype(jnp.float32)))
    total = jnp.sum(mse_ref[...]) / (H * W) / 1000.0 + loss_p + loss_t
    out_ref[...] = jnp.reshape(jnp.where(ninst == 0.0, 0.0, total), (1, 1))


def kernel(pred_mask, true_mask):
    hist, msep = _sc_hist(pred_mask.reshape(NPIX), true_mask.reshape(NPIX))
    out = pl.pallas_call(
        _tc_epilogue_kernel,
        out_shape=jax.ShapeDtypeStruct((1, 1), jnp.float32),
    )(hist, msep)
    return out[0, 0]


# speedup vs baseline: 1.5065x; 1.5065x over previous
import functools

import jax
import jax.numpy as jnp
from jax import lax
from jax.experimental import pallas as pl
from jax.experimental.pallas import tpu as pltpu
from jax.experimental.pallas import tpu_sc as plsc

NUM = 16
H = 1024
W = 1024
NPIX = H * W

_info = plsc.get_sparse_core_info()
NC, NS, L = _info.num_cores, _info.num_subcores, _info.num_lanes
NW = NC * NS
PPW = NPIX // NW
NBINS = NUM * NUM
UNROLL = 8


def _sc_hist_kernel(pred_hbm, true_hbm, hist_out, pbuf, tbuf, hacc, rowbuf):
    wid = lax.axis_index("s") * NC + lax.axis_index("c")
    base = wid * PPW

    pltpu.sync_copy(pred_hbm.at[pl.ds(base, PPW)], pbuf)
    pltpu.sync_copy(true_hbm.at[pl.ds(base, PPW)], tbuf)

    zero16 = jnp.zeros((L,), jnp.float32)

    def zbody(i, _):
        hacc[pl.ds(pl.multiple_of(i * L, L), L)] = zero16
        return 0
    lax.fori_loop(0, (L * NBINS) // L, zbody, 0)

    lane_offs = lax.iota(jnp.int32, L) * NBINS
    ones = jnp.ones((L,), jnp.float32)

    @plsc.parallel_loop(0, PPW, L, unroll=UNROLL)
    def body(i):
        off = pl.multiple_of(i, L)
        p = pbuf[pl.ds(off, L)]
        t = tbuf[pl.ds(off, L)]
        comb = (p * 16.0 + t).astype(jnp.int32) + lane_offs
        plsc.addupdate_scatter(hacc, [comb], ones)

    for c in range(NBINS // L):
        def mbody(l, acc, c=c):
            return acc + hacc[pl.ds(pl.multiple_of(l * NBINS + c * L, L), L)]
        rowbuf[pl.ds(c * L, L)] = lax.fori_loop(0, L, mbody, zero16)

    pltpu.sync_copy(rowbuf, hist_out.at[wid])


_sc_hist = functools.partial(
    pl.kernel,
    mesh=plsc.VectorSubcoreMesh(core_axis_name="c", subcore_axis_name="s"),
    out_type=jax.ShapeDtypeStruct((NW, NBINS), jnp.float32),
    scratch_types=[pltpu.VMEM((PPW,), jnp.float32),
                   pltpu.VMEM((PPW,), jnp.float32),
                   pltpu.VMEM((L * NBINS,), jnp.float32),
                   pltpu.VMEM((NBINS,), jnp.float32)],
    compiler_params=pltpu.CompilerParams(needs_layout_passes=False),
)(_sc_hist_kernel)


def _tc_epilogue_kernel(hist_ref, out_ref):
    h = hist_ref[...]
    flat = jnp.sum(h, axis=0, keepdims=True)

    bi = jax.lax.broadcasted_iota(jnp.int32, (NUM, NBINS), 0)
    bm = jax.lax.broadcasted_iota(jnp.int32, (NUM, NBINS), 1)
    d = jnp.where((bm >> 4) == bi, flat, 0.0)
    cm = jax.lax.broadcasted_iota(jnp.int32, (NBINS, NUM), 0)
    cj = jax.lax.broadcasted_iota(jnp.int32, (NBINS, NUM), 1)
    c = ((cm & (NUM - 1)) == cj).astype(jnp.float32)
    j = jax.lax.dot_general(d, c, (((1,), (0,)), ((), ())),
                            preferred_element_type=jnp.float32)

    ri = jax.lax.broadcasted_iota(jnp.int32, (NUM, NUM), 0)
    ci = jax.lax.broadcasted_iota(jnp.int32, (NUM, NUM), 1)
    df = (ri - ci).astype(jnp.float32)
    mse_sum = jnp.sum(j * df * df)
    valid = (ri >= 1) & (ci >= 1)
    inter = jnp.where(valid, j, 0.0)
    pc = jnp.sum(j, axis=1, keepdims=True)
    tc = jnp.sum(j, axis=0, keepdims=True)
    union = pc + tc - inter
    iou = jnp.where(valid & (union != 0.0),
                    inter / jnp.maximum(union, 1e-12), 0.0)
    max_p = jnp.max(iou, axis=1, keepdims=True)
    max_t = jnp.max(iou, axis=0, keepdims=True)
    rv = (jax.lax.broadcasted_iota(jnp.int32, (NUM, 1), 0) >= 1) & (pc > 0)
    cv = (jax.lax.broadcasted_iota(jnp.int32, (1, NUM), 1) >= 1) & (tc > 0)
    loss_p = jnp.sum(jnp.where(rv, 1.0 - max_p, 0.0))
    loss_t = jnp.sum(jnp.where(cv, 1.0 - max_t, 0.0))
    ninst = (jnp.sum(rv.astype(jnp.float32))
             + jnp.sum(cv.astype(jnp.float32)))
    total = mse_sum / (H * W) / 1000.0 + loss_p + loss_t
    out_ref[...] = jnp.reshape(jnp.where(ninst == 0.0, 0.0, total), (1, 1))


def kernel(pred_mask, true_mask):
    hist = _sc_hist(pred_mask.reshape(NPIX), true_mask.reshape(NPIX))
    out = pl.pallas_call(
        _tc_epilogue_kernel,
        out_shape=jax.ShapeDtypeStruct((1, 1), jnp.float32),
    )(hist)
    return out[0, 0]


# speedup vs baseline: 1.7359x; 1.1523x over previous
import functools

import jax
import jax.numpy as jnp
from jax import lax
from jax.experimental import pallas as pl
from jax.experimental.pallas import tpu as pltpu
from jax.experimental.pallas import tpu_sc as plsc

NUM = 16
H = 1024
W = 1024
NPIX = H * W

_info = plsc.get_sparse_core_info()
NC, NS, L = _info.num_cores, _info.num_subcores, _info.num_lanes
NW = NC * NS
PPW = NPIX // NW
NBINS = NUM * NUM
UNROLL = 8
RPW = H // NW
VPR = W // 16
VPR_LOG2 = 6


def _sc_hist_kernel(pred_hbm, true_hbm, hist_out, pbuf, tbuf, hacc, rowbuf):
    wid = lax.axis_index("s") * NC + lax.axis_index("c")

    pltpu.sync_copy(pred_hbm.at[pl.ds(wid * RPW, RPW)], pbuf)
    pltpu.sync_copy(true_hbm.at[pl.ds(wid * RPW, RPW)], tbuf)

    zero16 = jnp.zeros((L,), jnp.float32)

    def zbody(i, _):
        hacc[pl.ds(pl.multiple_of(i * L, L), L)] = zero16
        return 0
    lax.fori_loop(0, (L * NBINS) // L, zbody, 0)

    lane_offs = lax.iota(jnp.int32, L) * NBINS
    ones = jnp.ones((L,), jnp.float32)

    @plsc.parallel_loop(0, PPW // L, 1, unroll=UNROLL)
    def body(i):
        r = i >> VPR_LOG2
        c = pl.multiple_of((i & (VPR - 1)) * L, L)
        p = pbuf[r, pl.ds(c, L)]
        t = tbuf[r, pl.ds(c, L)]
        comb = (p * 16.0 + t).astype(jnp.int32) + lane_offs
        plsc.addupdate_scatter(hacc, [comb], ones)

    for c in range(NBINS // L):
        def mbody(l, acc, c=c):
            return acc + hacc[pl.ds(pl.multiple_of(l * NBINS + c * L, L), L)]
        rowbuf[pl.ds(c * L, L)] = lax.fori_loop(0, L, mbody, zero16)

    pltpu.sync_copy(rowbuf, hist_out.at[wid])


_sc_hist = functools.partial(
    pl.kernel,
    mesh=plsc.VectorSubcoreMesh(core_axis_name="c", subcore_axis_name="s"),
    out_type=jax.ShapeDtypeStruct((NW, NBINS), jnp.float32),
    scratch_types=[pltpu.VMEM((RPW, W), jnp.float32),
                   pltpu.VMEM((RPW, W), jnp.float32),
                   pltpu.VMEM((L * NBINS,), jnp.float32),
                   pltpu.VMEM((NBINS,), jnp.float32)],
    compiler_params=pltpu.CompilerParams(needs_layout_passes=False),
)(_sc_hist_kernel)


def _tc_epilogue_kernel(hist_ref, out_ref):
    h = hist_ref[...]
    flat = jnp.sum(h, axis=0, keepdims=True)

    bi = jax.lax.broadcasted_iota(jnp.int32, (NUM, NBINS), 0)
    bm = jax.lax.broadcasted_iota(jnp.int32, (NUM, NBINS), 1)
    d = jnp.where((bm >> 4) == bi, flat, 0.0)
    cm = jax.lax.broadcasted_iota(jnp.int32, (NBINS, NUM), 0)
    cj = jax.lax.broadcasted_iota(jnp.int32, (NBINS, NUM), 1)
    c = ((cm & (NUM - 1)) == cj).astype(jnp.float32)
    j = jax.lax.dot_general(d, c, (((1,), (0,)), ((), ())),
                            preferred_element_type=jnp.float32)

    ri = jax.lax.broadcasted_iota(jnp.int32, (NUM, NUM), 0)
    ci = jax.lax.broadcasted_iota(jnp.int32, (NUM, NUM), 1)
    df = (ri - ci).astype(jnp.float32)
    mse_sum = jnp.sum(j * df * df)
    valid = (ri >= 1) & (ci >= 1)
    inter = jnp.where(valid, j, 0.0)
    pc = jnp.sum(j, axis=1, keepdims=True)
    tc = jnp.sum(j, axis=0, keepdims=True)
    union = pc + tc - inter
    iou = jnp.where(valid & (union != 0.0),
                    inter / jnp.maximum(union, 1e-12), 0.0)
    max_p = jnp.max(iou, axis=1, keepdims=True)
    max_t = jnp.max(iou, axis=0, keepdims=True)
    rv = (jax.lax.broadcasted_iota(jnp.int32, (NUM, 1), 0) >= 1) & (pc > 0)
    cv = (jax.lax.broadcasted_iota(jnp.int32, (1, NUM), 1) >= 1) & (tc > 0)
    loss_p = jnp.sum(jnp.where(rv, 1.0 - max_p, 0.0))
    loss_t = jnp.sum(jnp.where(cv, 1.0 - max_t, 0.0))
    ninst = (jnp.sum(rv.astype(jnp.float32))
             + jnp.sum(cv.astype(jnp.float32)))
    total = mse_sum / (H * W) / 1000.0 + loss_p + loss_t
    out_ref[...] = jnp.reshape(jnp.where(ninst == 0.0, 0.0, total), (1, 1))


def kernel(pred_mask, true_mask):
    hist = _sc_hist(pred_mask, true_mask)
    out = pl.pallas_call(
        _tc_epilogue_kernel,
        out_shape=jax.ShapeDtypeStruct((1, 1), jnp.float32),
    )(hist)
    return out[0, 0]


# speedup vs baseline: 1.9355x; 1.1150x over previous
import functools

import jax
import jax.numpy as jnp
from jax import lax
from jax.experimental import pallas as pl
from jax.experimental.pallas import tpu as pltpu
from jax.experimental.pallas import tpu_sc as plsc

NUM = 16
H = 1024
W = 1024
NPIX = H * W

_info = plsc.get_sparse_core_info()
NC, NS, L = _info.num_cores, _info.num_subcores, _info.num_lanes
NW = NC * NS
PPW = NPIX // NW
NBINS = NUM * NUM
UNROLL = 8
RPW = H // NW
VPR = W // 16
VPR_LOG2 = 6
NCH = 4
CR = RPW // NCH


def _sc_hist_kernel(pred_hbm, true_hbm, hist_out,
                    pbuf0, tbuf0, pbuf1, tbuf1, hacc, rowbuf,
                    sp0, st0, sp1, st1):
    wid = lax.axis_index("s") * NC + lax.axis_index("c")
    pbufs, tbufs = (pbuf0, pbuf1), (tbuf0, tbuf1)
    sems = ((sp0, st0), (sp1, st1))

    def start(k, slot):
        row = wid * RPW + k * CR
        hp = pltpu.async_copy(pred_hbm.at[pl.ds(row, CR)], pbufs[slot],
                              sems[slot][0])
        ht = pltpu.async_copy(true_hbm.at[pl.ds(row, CR)], tbufs[slot],
                              sems[slot][1])
        return hp, ht

    inflight = [None, None]
    inflight[0] = start(0, 0)

    zero16 = jnp.zeros((L,), jnp.float32)

    def zbody(i, _):
        hacc[pl.ds(pl.multiple_of(i * L, L), L)] = zero16
        return 0
    lax.fori_loop(0, (L * NBINS) // L, zbody, 0)

    lane_offs = lax.iota(jnp.int32, L) * NBINS
    ones = jnp.ones((L,), jnp.float32)

    for k in range(NCH):
        slot = k % 2
        if k + 1 < NCH:
            inflight[1 - slot] = start(k + 1, 1 - slot)
        hp, ht = inflight[slot]
        hp.wait()
        ht.wait()
        pb, tb = pbufs[slot], tbufs[slot]

        @plsc.parallel_loop(0, (CR * W) // L, 1, unroll=UNROLL)
        def body(i, pb=pb, tb=tb):
            r = i >> VPR_LOG2
            c = pl.multiple_of((i & (VPR - 1)) * L, L)
            p = pb[r, pl.ds(c, L)]
            t = tb[r, pl.ds(c, L)]
            comb = (p * 16.0 + t).astype(jnp.int32) + lane_offs
            plsc.addupdate_scatter(hacc, [comb], ones)

    for c in range(NBINS // L):
        def mbody(l, acc, c=c):
            return acc + hacc[pl.ds(pl.multiple_of(l * NBINS + c * L, L), L)]
        rowbuf[pl.ds(c * L, L)] = lax.fori_loop(0, L, mbody, zero16)

    pltpu.sync_copy(rowbuf, hist_out.at[wid])


_sc_hist = functools.partial(
    pl.kernel,
    mesh=plsc.VectorSubcoreMesh(core_axis_name="c", subcore_axis_name="s"),
    out_type=jax.ShapeDtypeStruct((NW, NBINS), jnp.float32),
    scratch_types=[pltpu.VMEM((CR, W), jnp.float32),
                   pltpu.VMEM((CR, W), jnp.float32),
                   pltpu.VMEM((CR, W), jnp.float32),
                   pltpu.VMEM((CR, W), jnp.float32),
                   pltpu.VMEM((L * NBINS,), jnp.float32),
                   pltpu.VMEM((NBINS,), jnp.float32),
                   pltpu.SemaphoreType.DMA,
                   pltpu.SemaphoreType.DMA,
                   pltpu.SemaphoreType.DMA,
                   pltpu.SemaphoreType.DMA],
    compiler_params=pltpu.CompilerParams(needs_layout_passes=False),
)(_sc_hist_kernel)


def _tc_epilogue_kernel(hist_ref, out_ref):
    h = hist_ref[...]
    flat = jnp.sum(h, axis=0, keepdims=True)

    bi = jax.lax.broadcasted_iota(jnp.int32, (NUM, NBINS), 0)
    bm = jax.lax.broadcasted_iota(jnp.int32, (NUM, NBINS), 1)
    d = jnp.where((bm >> 4) == bi, flat, 0.0)
    cm = jax.lax.broadcasted_iota(jnp.int32, (NBINS, NUM), 0)
    cj = jax.lax.broadcasted_iota(jnp.int32, (NBINS, NUM), 1)
    c = ((cm & (NUM - 1)) == cj).astype(jnp.float32)
    j = jax.lax.dot_general(d, c, (((1,), (0,)), ((), ())),
                            preferred_element_type=jnp.float32)

    ri = jax.lax.broadcasted_iota(jnp.int32, (NUM, NUM), 0)
    ci = jax.lax.broadcasted_iota(jnp.int32, (NUM, NUM), 1)
    df = (ri - ci).astype(jnp.float32)
    mse_sum = jnp.sum(j * df * df)
    valid = (ri >= 1) & (ci >= 1)
    inter = jnp.where(valid, j, 0.0)
    pc = jnp.sum(j, axis=1, keepdims=True)
    tc = jnp.sum(j, axis=0, keepdims=True)
    union = pc + tc - inter
    iou = jnp.where(valid & (union != 0.0),
                    inter / jnp.maximum(union, 1e-12), 0.0)
    max_p = jnp.max(iou, axis=1, keepdims=True)
    max_t = jnp.max(iou, axis=0, keepdims=True)
    rv = (jax.lax.broadcasted_iota(jnp.int32, (NUM, 1), 0) >= 1) & (pc > 0)
    cv = (jax.lax.broadcasted_iota(jnp.int32, (1, NUM), 1) >= 1) & (tc > 0)
    loss_p = jnp.sum(jnp.where(rv, 1.0 - max_p, 0.0))
    loss_t = jnp.sum(jnp.where(cv, 1.0 - max_t, 0.0))
    ninst = (jnp.sum(rv.astype(jnp.float32))
             + jnp.sum(cv.astype(jnp.float32)))
    total = mse_sum / (H * W) / 1000.0 + loss_p + loss_t
    out_ref[...] = jnp.reshape(jnp.where(ninst == 0.0, 0.0, total), (1, 1))


def kernel(pred_mask, true_mask):
    hist = _sc_hist(pred_mask, true_mask)
    out = pl.pallas_call(
        _tc_epilogue_kernel,
        out_shape=jax.ShapeDtypeStruct((1, 1), jnp.float32),
    )(hist)
    return out[0, 0]
